# baseline (speedup 1.0000x reference)
import functools

import jax
import jax.numpy as jnp
from jax import lax
from jax.experimental import pallas as pl
from jax.experimental.pallas import tpu as pltpu
from jax.experimental.pallas import tpu_sc as plsc

N_VOCAB = 1000000
N_EMBED = 64
BATCH = 16384
N_SAMPLES = 5

NC = 2
NS = 16
NW = NC * NS
CHUNK = 128
NBUF = 4

IN_CH = BATCH // (NW * CHUNK)
NZ_CH = BATCH * N_SAMPLES // (NW * CHUNK)

_mesh = plsc.VectorSubcoreMesh(core_axis_name="c", subcore_axis_name="s")


@functools.partial(
    pl.kernel,
    mesh=_mesh,
    compiler_params=pltpu.CompilerParams(needs_layout_passes=False),
    out_type=[
        jax.ShapeDtypeStruct((BATCH, N_EMBED), jnp.float32),
        jax.ShapeDtypeStruct((BATCH, N_EMBED), jnp.float32),
        jax.ShapeDtypeStruct((BATCH * N_SAMPLES, N_EMBED), jnp.float32),
    ],
    scratch_types=[
        pltpu.VMEM((IN_CH, CHUNK), jnp.int32),
        pltpu.VMEM((IN_CH, CHUNK), jnp.int32),
        pltpu.VMEM((NZ_CH, CHUNK), jnp.int32),
        pltpu.VMEM((NBUF, CHUNK, N_EMBED), jnp.float32),
        pltpu.SemaphoreType.DMA,
        pltpu.SemaphoreType.DMA,
        pltpu.SemaphoreType.DMA,
        pltpu.SemaphoreType.DMA,
        pltpu.SemaphoreType.DMA,
        pltpu.SemaphoreType.DMA,
        pltpu.SemaphoreType.DMA,
        pltpu.SemaphoreType.DMA,
    ],
)
def _gather3(in_tab, out_tab, idx_in, idx_out, idx_nz,
             o_in, o_out, o_nz,
             wi, wo, wn, bufs, *sems):
    gsem = sems[:NBUF]
    ssem = sems[NBUF:]
    w = lax.axis_index("s") * NC + lax.axis_index("c")
    pltpu.sync_copy(idx_in.at[w], wi)
    pltpu.sync_copy(idx_out.at[w], wo)
    pltpu.sync_copy(idx_nz.at[w], wn)

    def run_task(tab, words, out, nch, wbase):
        def issue_rows(slot, j):
            def group(g, carry):
                wv = words[j, pl.ds(g * 16, 16)]
                for m in range(16):
                    pltpu.async_copy(tab.at[wv[m]],
                                     bufs.at[slot, g * 16 + m], gsem[slot])
                return carry
            lax.fori_loop(0, CHUNK // 16, group, 0)

        def drain_rows(slot, j):
            pltpu.make_async_copy(
                out.at[pl.ds(wbase + j * CHUNK, CHUNK)], bufs.at[slot],
                gsem[slot]).wait()

        def s_desc(slot, j):
            return pltpu.make_async_copy(
                bufs.at[slot], out.at[pl.ds(wbase + j * CHUNK, CHUNK)],
                ssem[slot])

        for b in range(2):
            issue_rows(b, b)

        def body(i, carry):
            for b in range(NBUF):
                j = i * NBUF + b

                @pl.when(j - 2 >= 0)
                def _():
                    s_desc((b + 2) % NBUF, j - 2).wait()

                @pl.when(j + 2 < nch)
                def _():
                    issue_rows((b + 2) % NBUF, j + 2)

                drain_rows(b, j)
                s_desc(b, j).start()
            return carry

        lax.fori_loop(0, nch // NBUF, body, 0)
        s_desc((nch - 2) % NBUF, nch - 2).wait()
        s_desc((nch - 1) % NBUF, nch - 1).wait()

    run_task(in_tab, wi, o_in, IN_CH, w * IN_CH * CHUNK)
    run_task(out_tab, wo, o_out, IN_CH, w * IN_CH * CHUNK)
    run_task(out_tab, wn, o_nz, NZ_CH, w * NZ_CH * CHUNK)


def kernel(in_embed_weight, out_embed_weight, input_words, output_words, noise_words):
    idx_in = input_words.astype(jnp.int32).reshape(NW, IN_CH, CHUNK)
    idx_out = output_words.astype(jnp.int32).reshape(NW, IN_CH, CHUNK)
    idx_nz = noise_words.astype(jnp.int32).reshape(NW, NZ_CH, CHUNK)
    o_in, o_out, o_nz = _gather3(
        in_embed_weight, out_embed_weight, idx_in, idx_out, idx_nz)
    return (o_in, o_out, o_nz.reshape(BATCH, N_SAMPLES, N_EMBED))
